# baseline (speedup 1.0000x reference)
import jax
import jax.numpy as jnp
from jax import lax
from jax.experimental import pallas as pl
from jax.experimental.pallas import tpu as pltpu
from jax.experimental.pallas import tpu_sc as plsc

EPS_ = 1e-12

N_ROWS = 4096
N_COLS = 1000
NC = 2
NS = 16
NW = NC * NS
ROWS_PER_W = N_ROWS // NW
GROUPS = ROWS_PER_W // 16
NSTRIPE = 4
STRIPE = N_COLS // NSTRIPE
UNROLL = 2


def _insert3(t1, t2, t3, v):
    lo = jnp.minimum(t1, v)
    t1 = jnp.maximum(t1, v)
    lo2 = jnp.minimum(t2, lo)
    t2 = jnp.maximum(t2, lo)
    t3 = jnp.maximum(t3, lo2)
    return t1, t2, t3


def _redlr_body(x_hbm, y_hbm, out_hbm, xb0, xb1, yv, ov, sem0, sem1):
    wid = lax.axis_index("s") * NC + lax.axis_index("c")
    base = wid * ROWS_PER_W

    pltpu.sync_copy(y_hbm.at[pl.ds(base, ROWS_PER_W)], yv)

    bufs = [xb0, xb1]
    sems = [sem0, sem1]
    cps = [None, None]
    cps[0] = pltpu.async_copy(x_hbm.at[pl.ds(base, 16), :], xb0, sem0)

    row_iota = lax.iota(jnp.int32, 16)
    neg = jnp.full((16,), -jnp.inf, jnp.float32)

    for g in range(GROUPS):
        b = g % 2
        if g + 1 < GROUPS:
            nb = (g + 1) % 2
            cps[nb] = pltpu.async_copy(
                x_hbm.at[pl.ds(base + (g + 1) * 16, 16), :], bufs[nb], sems[nb])
        cps[b].wait()
        xb = bufs[b]

        def col_body(c, carry, xb=xb):
            ts = list(carry[:-1])
            colv = carry[-1]
            for u in range(UNROLL):
                for k in range(NSTRIPE):
                    off = k * STRIPE + u
                    idx = colv + jnp.int32(off) if off else colv
                    v = plsc.load_gather(xb, [row_iota, idx])
                    i0 = 3 * k
                    ts[i0], ts[i0 + 1], ts[i0 + 2] = _insert3(
                        ts[i0], ts[i0 + 1], ts[i0 + 2], v)
            return (*ts, colv + jnp.int32(UNROLL))

        init = tuple([neg] * (3 * NSTRIPE)) + (jnp.zeros((16,), jnp.int32),)
        res = lax.fori_loop(0, STRIPE // UNROLL, col_body, init)

        t1, t2, t3 = res[0], res[1], res[2]
        for k in range(1, NSTRIPE):
            for j in range(3):
                t1, t2, t3 = _insert3(t1, t2, t3, res[3 * k + j])

        ylane = yv[pl.ds(g * 16, 16)]
        xy = plsc.load_gather(xb, [row_iota, ylane])
        indf = jnp.where(xy == t1, jnp.float32(1.0), jnp.float32(0.0))
        out = -(xy - t2) / (t1 - t3 + jnp.float32(EPS_)) * indf
        ov[pl.ds(g * 16, 16)] = out

    pltpu.sync_copy(ov, out_hbm.at[pl.ds(base, ROWS_PER_W)])


@jax.jit
def _redlr(x, y):
    mesh = plsc.VectorSubcoreMesh(core_axis_name="c", subcore_axis_name="s")
    return pl.kernel(
        _redlr_body,
        mesh=mesh,
        compiler_params=pltpu.CompilerParams(
            use_tc_tiling_on_sc=True, needs_layout_passes=False),
        out_type=jax.ShapeDtypeStruct((N_ROWS,), jnp.float32),
        scratch_types=[
            pltpu.VMEM((16, N_COLS), jnp.float32),
            pltpu.VMEM((16, N_COLS), jnp.float32),
            pltpu.VMEM((ROWS_PER_W,), jnp.int32),
            pltpu.VMEM((ROWS_PER_W,), jnp.float32),
            pltpu.SemaphoreType.DMA,
            pltpu.SemaphoreType.DMA,
        ],
    )(x, y)


def kernel(x, y):
    return _redlr(x, y.astype(jnp.int32))


# speedup vs baseline: 1.3417x; 1.3417x over previous
import jax
import jax.numpy as jnp
from jax import lax
from jax.experimental import pallas as pl
from jax.experimental.pallas import tpu as pltpu
from jax.experimental.pallas import tpu_sc as plsc

EPS_ = 1e-12

N_ROWS = 4096
N_COLS = 1000
NC = 2
NS = 16
NW = NC * NS
ROWS_PER_W = N_ROWS // NW
GROUPS = ROWS_PER_W // 16
NSTRIPE = 4
STRIPE = N_COLS // NSTRIPE
UNROLL = 2


def _insert3(t1, t2, t3, v):
    lo = jnp.minimum(t1, v)
    t1 = jnp.maximum(t1, v)
    lo2 = jnp.minimum(t2, lo)
    t2 = jnp.maximum(t2, lo)
    t3 = jnp.maximum(t3, lo2)
    return t1, t2, t3


def _redlr_body(x_hbm, y_hbm, out_hbm, xb0, xb1, yv, ov, sem0, sem1):
    wid = lax.axis_index("s") * NC + lax.axis_index("c")
    base = wid * ROWS_PER_W

    pltpu.sync_copy(y_hbm.at[pl.ds(base, ROWS_PER_W)], yv)

    bufs = [xb0, xb1]
    sems = [sem0, sem1]
    cps = [None, None]
    cps[0] = pltpu.async_copy(
        x_hbm.at[pl.ds(base * N_COLS, 16 * N_COLS)], xb0, sem0)

    row_iota = lax.iota(jnp.int32, 16)
    addr0 = row_iota * jnp.int32(N_COLS)
    neg = jnp.full((16,), -jnp.inf, jnp.float32)

    for g in range(GROUPS):
        b = g % 2
        if g + 1 < GROUPS:
            nb = (g + 1) % 2
            cps[nb] = pltpu.async_copy(
                x_hbm.at[pl.ds((base + (g + 1) * 16) * N_COLS, 16 * N_COLS)],
                bufs[nb], sems[nb])
        cps[b].wait()
        xb = bufs[b]

        def col_body(c, carry, xb=xb):
            ts = list(carry[:-1])
            addrv = carry[-1]
            for u in range(UNROLL):
                for k in range(NSTRIPE):
                    off = k * STRIPE + u
                    idx = addrv + jnp.int32(off) if off else addrv
                    v = plsc.load_gather(xb, [idx])
                    i0 = 3 * k
                    ts[i0], ts[i0 + 1], ts[i0 + 2] = _insert3(
                        ts[i0], ts[i0 + 1], ts[i0 + 2], v)
            return (*ts, addrv + jnp.int32(UNROLL))

        init = tuple([neg] * (3 * NSTRIPE)) + (addr0,)
        res = lax.fori_loop(0, STRIPE // UNROLL, col_body, init)

        t1, t2, t3 = res[0], res[1], res[2]
        for k in range(1, NSTRIPE):
            for j in range(3):
                t1, t2, t3 = _insert3(t1, t2, t3, res[3 * k + j])

        ylane = yv[pl.ds(g * 16, 16)]
        xy = plsc.load_gather(xb, [addr0 + ylane])
        indf = jnp.where(xy == t1, jnp.float32(1.0), jnp.float32(0.0))
        out = -(xy - t2) / (t1 - t3 + jnp.float32(EPS_)) * indf
        ov[pl.ds(g * 16, 16)] = out

    pltpu.sync_copy(ov, out_hbm.at[pl.ds(base, ROWS_PER_W)])


@jax.jit
def _redlr(x_flat, y):
    mesh = plsc.VectorSubcoreMesh(core_axis_name="c", subcore_axis_name="s")
    return pl.kernel(
        _redlr_body,
        mesh=mesh,
        compiler_params=pltpu.CompilerParams(
            use_tc_tiling_on_sc=False, needs_layout_passes=False),
        out_type=jax.ShapeDtypeStruct((N_ROWS,), jnp.float32),
        scratch_types=[
            pltpu.VMEM((16 * N_COLS,), jnp.float32),
            pltpu.VMEM((16 * N_COLS,), jnp.float32),
            pltpu.VMEM((ROWS_PER_W,), jnp.int32),
            pltpu.VMEM((ROWS_PER_W,), jnp.float32),
            pltpu.SemaphoreType.DMA,
            pltpu.SemaphoreType.DMA,
        ],
    )(x_flat, y)


def kernel(x, y):
    return _redlr(x.reshape(N_ROWS * N_COLS), y.astype(jnp.int32))
